# baseline (speedup 1.0000x reference)
import functools

import jax
import jax.numpy as jnp
from jax import lax
from jax.experimental import pallas as pl
from jax.experimental.pallas import tpu as pltpu
from jax.experimental.pallas import tpu_sc as plsc

_SEG = 2176
_WROW = 4992
_QB = 256


def _argmin_kernel(ang_ref, out_ref):
    ang = ang_ref[...]
    m = jnp.min(ang, axis=1, keepdims=True)
    d = ang.shape[1]
    iota = lax.broadcasted_iota(jnp.int32, ang.shape, 1)
    idx = jnp.min(jnp.where(ang == m, iota, d), axis=1)
    out_ref[...] = idx[:, None]


def _combine_kernel(g_ref, x_ref, s_ref, ds_ref, bf_ref, bw_ref):
    x = x_ref[...]
    wds = g_ref[:, _SEG:_SEG + 2056]
    ds_ref[...] = wds * x
    p = g_ref[:, 0:2056] * x
    bf_ref[...] = jax.lax.dot(p, s_ref[...],
                              preferred_element_type=jnp.float32)
    bw_ref[...] = g_ref[:, 2 * _SEG:2 * _SEG + 514]


def _tc_argmin(angle2d):
    q, d = angle2d.shape
    return pl.pallas_call(
        _argmin_kernel,
        grid=(q // _QB,),
        in_specs=[pl.BlockSpec((_QB, d), lambda i: (i, 0))],
        out_specs=pl.BlockSpec((_QB, 1), lambda i: (i, 0)),
        out_shape=jax.ShapeDtypeStruct((q, 1), jnp.int32),
        compiler_params=pltpu.CompilerParams(
            dimension_semantics=("parallel",)),
    )(angle2d)


def _tc_combine(g, x2d, sel):
    q = x2d.shape[0]
    fc = x2d.shape[1]
    f = sel.shape[1]
    return pl.pallas_call(
        _combine_kernel,
        grid=(q // _QB,),
        in_specs=[
            pl.BlockSpec((_QB, _WROW), lambda i: (i, 0)),
            pl.BlockSpec((_QB, fc), lambda i: (i, 0)),
            pl.BlockSpec(sel.shape, lambda i: (0, 0)),
        ],
        out_specs=[
            pl.BlockSpec((_QB, fc), lambda i: (i, 0)),
            pl.BlockSpec((_QB, f), lambda i: (i, 0)),
            pl.BlockSpec((_QB, 514), lambda i: (i, 0)),
        ],
        out_shape=[
            jax.ShapeDtypeStruct((q, fc), jnp.float32),
            jax.ShapeDtypeStruct((q, f), jnp.float32),
            jax.ShapeDtypeStruct((q, 514), jnp.float32),
        ],
        compiler_params=pltpu.CompilerParams(
            dimension_semantics=("parallel",)),
    )(g, x2d, sel)


def _sc_gather(table, idx3d, n_workers, n_chunks, chunk):
    q = n_workers * n_chunks * chunk
    mesh = plsc.VectorSubcoreMesh(core_axis_name="c", subcore_axis_name="s")
    num_cores = mesh.num_cores

    @functools.partial(
        pl.kernel,
        mesh=mesh,
        out_type=jax.ShapeDtypeStruct((q, _WROW), jnp.float32),
        scratch_types=[
            pltpu.VMEM((n_chunks, chunk), jnp.int32),
            pltpu.VMEM((chunk, _WROW), jnp.float32),
            pltpu.SemaphoreType.DMA,
        ],
    )
    def gather(table_hbm, idx_hbm, out_hbm, idx_v, rows_v, sem):
        wid = lax.axis_index("s") * num_cores + lax.axis_index("c")
        base = wid * (n_chunks * chunk)
        pltpu.sync_copy(idx_hbm.at[wid], idx_v)

        def body(k, carry):
            pltpu.async_copy(table_hbm.at[idx_v.at[k]], rows_v, sem).wait()
            pltpu.sync_copy(rows_v, out_hbm.at[pl.ds(base + k * chunk, chunk)])
            return carry

        lax.fori_loop(0, n_chunks, body, 0)

    return gather(table, idx3d)


def kernel(X, target_doas, dirs, w_conj, w_conj_ds, w_binaural):
    B, T, F, C = X.shape
    D = dirs.shape[0]
    O = w_binaural.shape[0]
    Q = B * T

    t = (jnp.pi / 180.0) * target_doas[:, :T, :]
    azi_diff = t[..., 0][:, :, None] - dirs[..., 0][None, None, :]
    zen_diff = t[..., 1][:, :, None] - dirs[..., 1][None, None, :]
    a = jnp.sin(zen_diff / 2.0) ** 2 + jnp.cos(t[..., 1][:, :, None]) * \
        jnp.cos(dirs[..., 1][None, None, :]) * jnp.sin(azi_diff / 2.0) ** 2
    angle = 2.0 * jnp.arcsin(jnp.sqrt(jnp.clip(a, 0.0, 1.0)))

    ind = _tc_argmin(angle.reshape(Q, D))

    wcT = jnp.transpose(w_conj, (2, 1, 0)).reshape(D, F * C)
    wdsT = jnp.transpose(w_conj_ds, (2, 1, 0)).reshape(D, F * C)
    wbT = jnp.transpose(w_binaural, (2, 1, 0)).reshape(D, F * O)
    zpad = jnp.zeros((D, _SEG - F * C), jnp.float32)
    packed = jnp.concatenate(
        [wcT, zpad, wdsT, zpad, wbT,
         jnp.zeros((D, _WROW - 2 * _SEG - F * O), jnp.float32)], axis=1)

    info = plsc.get_sparse_core_info()
    n_workers = info.num_cores * info.num_subcores
    chunk = 8
    n_chunks = Q // (n_workers * chunk)
    idx3d = ind.reshape(n_workers, n_chunks, chunk)

    g = _sc_gather(packed, idx3d, n_workers, n_chunks, chunk)

    sel = (jnp.arange(F * C, dtype=jnp.int32)[:, None] // C ==
           jnp.arange(F, dtype=jnp.int32)[None, :]).astype(jnp.float32)
    ds2, bf2, bw2 = _tc_combine(g, X.reshape(Q, F * C), sel)

    return (ds2.reshape(B, T, F, C), bf2.reshape(B, T, F),
            bw2.reshape(B, T, F, O))


# speedup vs baseline: 1.0699x; 1.0699x over previous
import jax
import jax.numpy as jnp
from jax import lax
from jax.experimental import pallas as pl
from jax.experimental.pallas import tpu as pltpu

_QB = 256


def _fused_kernel(ang_ref, x_ref, wc_ref, wds_ref, wb_ref, s_ref,
                  ds_ref, bf_ref, bw_ref):
    ang = ang_ref[...]
    m = jnp.min(ang, axis=1, keepdims=True)
    iota = lax.broadcasted_iota(jnp.int32, ang.shape, 1)
    idx = jnp.min(jnp.where(ang == m, iota, ang.shape[1]), axis=1,
                  keepdims=True)
    oh = (iota == idx).astype(jnp.bfloat16)

    nt = (((1,), (1,)), ((), ()))
    gc = lax.dot_general(oh, wc_ref[...], nt,
                         preferred_element_type=jnp.float32)
    gds = lax.dot_general(oh, wds_ref[...], nt,
                          preferred_element_type=jnp.float32)
    x = x_ref[...]
    ds_ref[...] = gds * x
    p = gc * x
    bf_ref[...] = jax.lax.dot(p, s_ref[...],
                              preferred_element_type=jnp.float32)
    bw_ref[...] = lax.dot_general(oh, wb_ref[...], nt,
                                  preferred_element_type=jnp.float32)


def kernel(X, target_doas, dirs, w_conj, w_conj_ds, w_binaural):
    B, T, F, C = X.shape
    D = dirs.shape[0]
    O = w_binaural.shape[0]
    Q = B * T
    FC = F * C

    t = (jnp.pi / 180.0) * target_doas[:, :T, :]
    azi_diff = t[..., 0][:, :, None] - dirs[..., 0][None, None, :]
    zen_diff = t[..., 1][:, :, None] - dirs[..., 1][None, None, :]
    a = jnp.sin(zen_diff / 2.0) ** 2 + jnp.cos(t[..., 1][:, :, None]) * \
        jnp.cos(dirs[..., 1][None, None, :]) * jnp.sin(azi_diff / 2.0) ** 2
    angle = 2.0 * jnp.arcsin(jnp.sqrt(jnp.clip(a, 0.0, 1.0)))
    angle2d = angle.reshape(Q, D)

    wc2 = jnp.transpose(w_conj, (1, 0, 2)).reshape(FC, D)
    wc2 = wc2.astype(jnp.bfloat16)
    wds2 = jnp.transpose(w_conj_ds, (1, 0, 2)).reshape(FC, D)
    wds2 = wds2.astype(jnp.bfloat16)
    wb2 = jnp.transpose(w_binaural, (1, 0, 2)).reshape(F * O, D)
    wb2 = wb2.astype(jnp.bfloat16)

    sel = (jnp.arange(FC, dtype=jnp.int32)[:, None] // C ==
           jnp.arange(F, dtype=jnp.int32)[None, :]).astype(jnp.float32)

    ds2, bf2, bw2 = pl.pallas_call(
        _fused_kernel,
        grid=(Q // _QB,),
        in_specs=[
            pl.BlockSpec((_QB, D), lambda i: (i, 0)),
            pl.BlockSpec((_QB, FC), lambda i: (i, 0)),
            pl.BlockSpec((FC, D), lambda i: (0, 0)),
            pl.BlockSpec((FC, D), lambda i: (0, 0)),
            pl.BlockSpec((F * O, D), lambda i: (0, 0)),
            pl.BlockSpec((FC, F), lambda i: (0, 0)),
        ],
        out_specs=[
            pl.BlockSpec((_QB, FC), lambda i: (i, 0)),
            pl.BlockSpec((_QB, F), lambda i: (i, 0)),
            pl.BlockSpec((_QB, F * O), lambda i: (i, 0)),
        ],
        out_shape=[
            jax.ShapeDtypeStruct((Q, FC), jnp.float32),
            jax.ShapeDtypeStruct((Q, F), jnp.float32),
            jax.ShapeDtypeStruct((Q, F * O), jnp.float32),
        ],
        compiler_params=pltpu.CompilerParams(
            dimension_semantics=("parallel",)),
    )(angle2d, X.reshape(Q, FC), wc2, wds2, wb2, sel)

    return (ds2.reshape(B, T, F, C), bf2.reshape(B, T, F),
            bw2.reshape(B, T, F, O))


# speedup vs baseline: 1.1445x; 1.0697x over previous
import jax
import jax.numpy as jnp
from jax import lax
from jax.experimental import pallas as pl
from jax.experimental.pallas import tpu as pltpu

_QB = 256


def _fused_kernel(ta_ref, tz_ref, da_ref, dz_ref, x_ref,
                  wc_ref, wds_ref, wb_ref, s_ref,
                  ds_ref, bf_ref, bw_ref):
    ta = ta_ref[...]
    tz = tz_ref[...]
    da = da_ref[...]
    dz = dz_ref[...]
    sz = jnp.sin((tz - dz) / 2.0)
    sa = jnp.sin((ta - da) / 2.0)
    a = sz ** 2 + jnp.cos(tz) * jnp.cos(dz) * sa ** 2
    ang = jnp.clip(a, 0.0, 1.0)
    m = jnp.min(ang, axis=1, keepdims=True)
    iota = lax.broadcasted_iota(jnp.int32, ang.shape, 1)
    idx = jnp.min(jnp.where(ang == m, iota, ang.shape[1]), axis=1,
                  keepdims=True)
    oh = (iota == idx).astype(jnp.bfloat16)

    nt = (((1,), (1,)), ((), ()))
    gc = lax.dot_general(oh, wc_ref[...], nt,
                         preferred_element_type=jnp.float32)
    gds = lax.dot_general(oh, wds_ref[...], nt,
                          preferred_element_type=jnp.float32)
    x = x_ref[...]
    ds_ref[...] = gds * x
    p = gc * x
    bf_ref[...] = jax.lax.dot(p, s_ref[...],
                              preferred_element_type=jnp.float32)
    bw_ref[...] = lax.dot_general(oh, wb_ref[...], nt,
                                  preferred_element_type=jnp.float32)


def kernel(X, target_doas, dirs, w_conj, w_conj_ds, w_binaural):
    B, T, F, C = X.shape
    D = dirs.shape[0]
    O = w_binaural.shape[0]
    Q = B * T
    FC = F * C

    t = (jnp.pi / 180.0) * target_doas[:, :T, :]
    ta = t[..., 0].reshape(Q, 1)
    tz = t[..., 1].reshape(Q, 1)
    da = dirs[..., 0].reshape(1, D)
    dz = dirs[..., 1].reshape(1, D)

    wc2 = jnp.transpose(w_conj, (1, 0, 2)).reshape(FC, D)
    wc2 = wc2.astype(jnp.bfloat16)
    wds2 = jnp.transpose(w_conj_ds, (1, 0, 2)).reshape(FC, D)
    wds2 = wds2.astype(jnp.bfloat16)
    wb2 = jnp.transpose(w_binaural, (1, 0, 2)).reshape(F * O, D)
    wb2 = wb2.astype(jnp.bfloat16)

    sel = (jnp.arange(FC, dtype=jnp.int32)[:, None] // C ==
           jnp.arange(F, dtype=jnp.int32)[None, :]).astype(jnp.float32)

    ds2, bf2, bw2 = pl.pallas_call(
        _fused_kernel,
        grid=(Q // _QB,),
        in_specs=[
            pl.BlockSpec((_QB, 1), lambda i: (i, 0)),
            pl.BlockSpec((_QB, 1), lambda i: (i, 0)),
            pl.BlockSpec((1, D), lambda i: (0, 0)),
            pl.BlockSpec((1, D), lambda i: (0, 0)),
            pl.BlockSpec((_QB, FC), lambda i: (i, 0)),
            pl.BlockSpec((FC, D), lambda i: (0, 0)),
            pl.BlockSpec((FC, D), lambda i: (0, 0)),
            pl.BlockSpec((F * O, D), lambda i: (0, 0)),
            pl.BlockSpec((FC, F), lambda i: (0, 0)),
        ],
        out_specs=[
            pl.BlockSpec((_QB, FC), lambda i: (i, 0)),
            pl.BlockSpec((_QB, F), lambda i: (i, 0)),
            pl.BlockSpec((_QB, F * O), lambda i: (i, 0)),
        ],
        out_shape=[
            jax.ShapeDtypeStruct((Q, FC), jnp.float32),
            jax.ShapeDtypeStruct((Q, F), jnp.float32),
            jax.ShapeDtypeStruct((Q, F * O), jnp.float32),
        ],
        compiler_params=pltpu.CompilerParams(
            dimension_semantics=("parallel",)),
    )(ta, tz, da, dz, X.reshape(Q, FC), wc2, wds2, wb2, sel)

    return (ds2.reshape(B, T, F, C), bf2.reshape(B, T, F),
            bw2.reshape(B, T, F, O))
